# Initial kernel scaffold; baseline (speedup 1.0000x reference)
import jax
import jax.numpy as jnp
from jax.experimental import pallas as pl


def kernel(h_g2, edge_index, W_lin, b_lin, W_ih, b_ih, W_hh, b_hh, gamma, beta):
    raise NotImplementedError("write your pallas kernel here")


# speedup vs baseline: 3.0420x; 3.0420x over previous
import functools

import jax
import jax.numpy as jnp
from jax import lax
from jax.experimental import pallas as pl
from jax.experimental.pallas import tpu as pltpu
from jax.experimental.pallas import tpu_sc as plsc

HIDDEN = 128
N_NODES = 10000
NC, NS = 2, 16
CH = 128
NCH = 80
IBLK = 16
NBLK = NCH // IBLK
E_PAD = NC * NS * NCH * CH
DUMMY = N_NODES
ACC_ROWS = 10240
ZROWS = ACC_ROWS // NS
OROWS = 624
ROW_BLK = 1000
F32 = jnp.float32

def _sc_edge_scatter_body(msg_hbm, src_hbm, dst_hbm, zeros_hbm, out_hbm,
                          src_v, dst_v, rows_a, rows_b, acc,
                          sem_a, sem_b, sem_s, sem_d):
    c = lax.axis_index("c")
    s = lax.axis_index("s")
    wid = c * NS + s

    pltpu.sync_copy(zeros_hbm, acc.at[pl.ds(s * ZROWS, ZROWS)])
    pltpu.async_copy(src_hbm.at[wid].at[pl.ds(0, IBLK)], src_v.at[0], sem_s)
    pltpu.async_copy(dst_hbm.at[wid].at[pl.ds(0, IBLK)], dst_v.at[0], sem_d)
    plsc.subcore_barrier()

    for b in range(NBLK):
        slot = b % 2
        pltpu.make_async_copy(src_hbm.at[wid].at[pl.ds(b * IBLK, IBLK)],
                              src_v.at[slot], sem_s).wait()
        pltpu.make_async_copy(dst_hbm.at[wid].at[pl.ds(b * IBLK, IBLK)],
                              dst_v.at[slot], sem_d).wait()
        if b + 1 < NBLK:
            nslot = (b + 1) % 2
            pltpu.async_copy(src_hbm.at[wid].at[pl.ds((b + 1) * IBLK, IBLK)],
                             src_v.at[nslot], sem_s)
            pltpu.async_copy(dst_hbm.at[wid].at[pl.ds((b + 1) * IBLK, IBLK)],
                             dst_v.at[nslot], sem_d)
        sv = src_v.at[slot]
        dv = dst_v.at[slot]

        pltpu.async_copy(msg_hbm.at[sv.at[0]], rows_a, sem_a)
        pltpu.async_copy(msg_hbm.at[sv.at[1]], rows_b, sem_b)

        @pl.loop(0, IBLK - 2, step=2)
        def _(j):
            pltpu.make_async_copy(msg_hbm.at[sv.at[j]], rows_a, sem_a).wait()
            pltpu.sync_copy(rows_a, acc.at[dv.at[j]], add=True)
            pltpu.async_copy(msg_hbm.at[sv.at[j + 2]], rows_a, sem_a)
            pltpu.make_async_copy(msg_hbm.at[sv.at[j + 1]], rows_b, sem_b).wait()
            pltpu.sync_copy(rows_b, acc.at[dv.at[j + 1]], add=True)
            pltpu.async_copy(msg_hbm.at[sv.at[j + 3]], rows_b, sem_b)

        pltpu.make_async_copy(msg_hbm.at[sv.at[IBLK - 2]], rows_a, sem_a).wait()
        pltpu.sync_copy(rows_a, acc.at[dv.at[IBLK - 2]], add=True)
        pltpu.make_async_copy(msg_hbm.at[sv.at[IBLK - 1]], rows_b, sem_b).wait()
        pltpu.sync_copy(rows_b, acc.at[dv.at[IBLK - 1]], add=True)

    plsc.subcore_barrier()
    pltpu.sync_copy(acc.at[pl.ds(s * OROWS, OROWS)],
                    out_hbm.at[c].at[pl.ds(s * OROWS, OROWS)])

    @pl.when(s == NS - 1)
    def _():
        pltpu.sync_copy(acc.at[pl.ds(NS * OROWS, N_NODES - NS * OROWS)],
                        out_hbm.at[c].at[pl.ds(NS * OROWS, N_NODES - NS * OROWS)])


@functools.cache
def _sc_edge_scatter_kernel():
    mesh = plsc.VectorSubcoreMesh(
        core_axis_name="c", subcore_axis_name="s",
        num_cores=NC, num_subcores=NS,
    )
    return pl.kernel(
        _sc_edge_scatter_body,
        out_type=jax.ShapeDtypeStruct((NC, N_NODES, HIDDEN), F32),
        mesh=mesh,
        scratch_types=[
            pltpu.VMEM((2, IBLK, CH), jnp.int32),
            pltpu.VMEM((2, IBLK, CH), jnp.int32),
            pltpu.VMEM((CH, HIDDEN), F32),
            pltpu.VMEM((CH, HIDDEN), F32),
            pltpu.VMEM_SHARED((ACC_ROWS, HIDDEN), F32),
            pltpu.SemaphoreType.DMA,
            pltpu.SemaphoreType.DMA,
            pltpu.SemaphoreType.DMA,
            pltpu.SemaphoreType.DMA,
        ],
    )


def _sc_edge_scatter(msg, src_p, dst_p, zeros):
    return _sc_edge_scatter_kernel()(msg, src_p, dst_p, zeros)


def _lin_body(x_ref, w_ref, b_ref, o_ref):
    o_ref[...] = (
        jnp.dot(x_ref[...], w_ref[...], preferred_element_type=F32) + b_ref[...]
    )


def _gru(a, h, wih_ref, bih_ref, whh_ref, bhh_ref):
    gi = jnp.dot(a, wih_ref[...], preferred_element_type=F32) + bih_ref[...]
    gh = jnp.dot(h, whh_ref[...], preferred_element_type=F32) + bhh_ref[...]
    r = jax.nn.sigmoid(gi[:, :HIDDEN] + gh[:, :HIDDEN])
    z = jax.nn.sigmoid(gi[:, HIDDEN:2 * HIDDEN] + gh[:, HIDDEN:2 * HIDDEN])
    n = jnp.tanh(gi[:, 2 * HIDDEN:] + r * gh[:, 2 * HIDDEN:])
    return (1.0 - z) * n + z * h


def _gru_lin_body(a0_ref, a1_ref, h_ref, wih_ref, bih_ref, whh_ref, bhh_ref,
                  wlin_ref, blin_ref, h_out, msg_out):
    a = a0_ref[0] + a1_ref[0]
    hn = _gru(a, h_ref[...], wih_ref, bih_ref, whh_ref, bhh_ref)
    h_out[...] = hn
    msg_out[...] = (
        jnp.dot(hn, wlin_ref[...], preferred_element_type=F32) + blin_ref[...]
    )


def _gru_ln_body(a0_ref, a1_ref, h_ref, res_ref, wih_ref, bih_ref, whh_ref,
                 bhh_ref, gamma_ref, beta_ref, o_ref):
    a = a0_ref[0] + a1_ref[0]
    hn = _gru(a, h_ref[...], wih_ref, bih_ref, whh_ref, bhh_ref)
    x = hn + res_ref[...]
    mean = jnp.mean(x, axis=-1, keepdims=True)
    xc = x - mean
    var = jnp.mean(xc * xc, axis=-1, keepdims=True)
    y = xc * lax.rsqrt(var + 1e-5)
    y = y * gamma_ref[...] + beta_ref[...]
    o_ref[...] = jnp.where(y >= 0, y, 0.01 * y)


def _row_spec(shape=(ROW_BLK, HIDDEN)):
    return pl.BlockSpec(shape, lambda i: (0,) * (len(shape) - 2) + (i, 0))


def _full_spec(shape):
    return pl.BlockSpec(shape, lambda i: (0,) * len(shape))


_GRID = N_NODES // ROW_BLK


def _tc_linear(x, w, b):
    return pl.pallas_call(
        _lin_body,
        grid=(_GRID,),
        in_specs=[_row_spec(), _full_spec((HIDDEN, HIDDEN)), _full_spec((1, HIDDEN))],
        out_specs=_row_spec(),
        out_shape=jax.ShapeDtypeStruct((N_NODES, HIDDEN), F32),
    )(x, w, b)


def _tc_gru_lin(parts, h, wih, bih, whh, bhh, wlin, blin):
    part_spec = pl.BlockSpec((1, ROW_BLK, HIDDEN), lambda i: (0, i, 0))
    part_spec1 = pl.BlockSpec((1, ROW_BLK, HIDDEN), lambda i: (1, i, 0))
    return pl.pallas_call(
        _gru_lin_body,
        grid=(_GRID,),
        in_specs=[part_spec, part_spec1, _row_spec(),
                  _full_spec((HIDDEN, 3 * HIDDEN)), _full_spec((1, 3 * HIDDEN)),
                  _full_spec((HIDDEN, 3 * HIDDEN)), _full_spec((1, 3 * HIDDEN)),
                  _full_spec((HIDDEN, HIDDEN)), _full_spec((1, HIDDEN))],
        out_specs=[_row_spec(), _row_spec()],
        out_shape=[jax.ShapeDtypeStruct((N_NODES, HIDDEN), F32),
                   jax.ShapeDtypeStruct((N_NODES, HIDDEN), F32)],
    )(parts, parts, h, wih, bih, whh, bhh, wlin, blin)


def _tc_gru_ln(parts, h, res, wih, bih, whh, bhh, gamma, beta):
    part_spec = pl.BlockSpec((1, ROW_BLK, HIDDEN), lambda i: (0, i, 0))
    part_spec1 = pl.BlockSpec((1, ROW_BLK, HIDDEN), lambda i: (1, i, 0))
    return pl.pallas_call(
        _gru_ln_body,
        grid=(_GRID,),
        in_specs=[part_spec, part_spec1, _row_spec(), _row_spec(),
                  _full_spec((HIDDEN, 3 * HIDDEN)), _full_spec((1, 3 * HIDDEN)),
                  _full_spec((HIDDEN, 3 * HIDDEN)), _full_spec((1, 3 * HIDDEN)),
                  _full_spec((1, HIDDEN)), _full_spec((1, HIDDEN))],
        out_specs=_row_spec(),
        out_shape=jax.ShapeDtypeStruct((N_NODES, HIDDEN), F32),
    )(parts, parts, h, res, wih, bih, whh, bhh, gamma, beta)


def kernel(h_g2, edge_index, W_lin, b_lin, W_ih, b_ih, W_hh, b_hh, gamma, beta):
    src = edge_index[0].astype(jnp.int32)
    dst = edge_index[1].astype(jnp.int32)
    e = src.shape[0]
    src_p = jnp.concatenate(
        [src, jnp.zeros((E_PAD - e,), jnp.int32)]).reshape(NC * NS, NCH, CH)
    dst_p = jnp.concatenate(
        [dst, jnp.full((E_PAD - e,), DUMMY, jnp.int32)]).reshape(NC * NS, NCH, CH)
    zeros = jnp.zeros((ZROWS, HIDDEN), F32)

    w_lin_t = W_lin.T
    w_ih_t = W_ih.T
    w_hh_t = W_hh.T
    b_lin2 = b_lin.reshape(1, HIDDEN)
    b_ih2 = b_ih.reshape(1, 3 * HIDDEN)
    b_hh2 = b_hh.reshape(1, 3 * HIDDEN)
    gamma2 = gamma.reshape(1, HIDDEN)
    beta2 = beta.reshape(1, HIDDEN)

    msg = _tc_linear(h_g2, w_lin_t, b_lin2)
    parts = _sc_edge_scatter(msg, src_p, dst_p, zeros)
    feat, msg2 = _tc_gru_lin(parts, h_g2, w_ih_t, b_ih2, w_hh_t, b_hh2,
                             w_lin_t, b_lin2)
    parts2 = _sc_edge_scatter(msg2, src_p, dst_p, zeros)
    out = _tc_gru_ln(parts2, feat, h_g2, w_ih_t, b_ih2, w_hh_t, b_hh2,
                     gamma2, beta2)
    return out


# speedup vs baseline: 3.2384x; 1.0646x over previous
import functools

import jax
import jax.numpy as jnp
from jax import lax
from jax.experimental import pallas as pl
from jax.experimental.pallas import tpu as pltpu
from jax.experimental.pallas import tpu_sc as plsc

HIDDEN = 128
N_NODES = 10000
NC, NS = 2, 16
CH = 128
IBLK = 16
NCH0 = 128
NCH1 = 32
TOT_CHUNKS = NS * (NCH0 + NCH1)
E_PAD = NS * (NCH0 + NCH1) * CH
DUMMY = N_NODES
ACC_ROWS = 10240
ZROWS = ACC_ROWS // NS
OROWS = 624
ROW_BLK = 1000
F32 = jnp.float32

def _sc_edge_scatter_body(msg_hbm, src_hbm, dst_hbm, zeros_hbm, out_hbm,
                          src_v, dst_v, rows_a, rows_b, acc,
                          sem_a, sem_b, sem_s, sem_d):
    c = lax.axis_index("c")
    s = lax.axis_index("s")

    pltpu.sync_copy(zeros_hbm, acc.at[pl.ds(s * ZROWS, ZROWS)])
    plsc.subcore_barrier()

    def process(base, nch):
        pltpu.async_copy(src_hbm.at[pl.ds(base, IBLK)], src_v.at[0], sem_s)
        pltpu.async_copy(dst_hbm.at[pl.ds(base, IBLK)], dst_v.at[0], sem_d)
        for b in range(nch // IBLK):
            slot = b % 2
            pltpu.make_async_copy(src_hbm.at[pl.ds(base + b * IBLK, IBLK)],
                                  src_v.at[slot], sem_s).wait()
            pltpu.make_async_copy(dst_hbm.at[pl.ds(base + b * IBLK, IBLK)],
                                  dst_v.at[slot], sem_d).wait()
            if b + 1 < nch // IBLK:
                nslot = (b + 1) % 2
                pltpu.async_copy(
                    src_hbm.at[pl.ds(base + (b + 1) * IBLK, IBLK)],
                    src_v.at[nslot], sem_s)
                pltpu.async_copy(
                    dst_hbm.at[pl.ds(base + (b + 1) * IBLK, IBLK)],
                    dst_v.at[nslot], sem_d)
            sv = src_v.at[slot]
            dv = dst_v.at[slot]

            pltpu.async_copy(msg_hbm.at[sv.at[0]], rows_a, sem_a)
            pltpu.async_copy(msg_hbm.at[sv.at[1]], rows_b, sem_b)

            @pl.loop(0, IBLK - 2, step=2)
            def _(j):
                pltpu.make_async_copy(msg_hbm.at[sv.at[j]], rows_a, sem_a).wait()
                pltpu.sync_copy(rows_a, acc.at[dv.at[j]], add=True)
                pltpu.async_copy(msg_hbm.at[sv.at[j + 2]], rows_a, sem_a)
                pltpu.make_async_copy(msg_hbm.at[sv.at[j + 1]], rows_b, sem_b).wait()
                pltpu.sync_copy(rows_b, acc.at[dv.at[j + 1]], add=True)
                pltpu.async_copy(msg_hbm.at[sv.at[j + 3]], rows_b, sem_b)

            pltpu.make_async_copy(msg_hbm.at[sv.at[IBLK - 2]], rows_a, sem_a).wait()
            pltpu.sync_copy(rows_a, acc.at[dv.at[IBLK - 2]], add=True)
            pltpu.make_async_copy(msg_hbm.at[sv.at[IBLK - 1]], rows_b, sem_b).wait()
            pltpu.sync_copy(rows_b, acc.at[dv.at[IBLK - 1]], add=True)

    @pl.when(c == 0)
    def _():
        process(s * NCH0, NCH0)

    @pl.when(c == 1)
    def _():
        process(NS * NCH0 + s * NCH1, NCH1)

    plsc.subcore_barrier()
    pltpu.sync_copy(acc.at[pl.ds(s * OROWS, OROWS)],
                    out_hbm.at[c].at[pl.ds(s * OROWS, OROWS)])

    @pl.when(s == NS - 1)
    def _():
        pltpu.sync_copy(acc.at[pl.ds(NS * OROWS, N_NODES - NS * OROWS)],
                        out_hbm.at[c].at[pl.ds(NS * OROWS, N_NODES - NS * OROWS)])


@functools.cache
def _sc_edge_scatter_kernel():
    mesh = plsc.VectorSubcoreMesh(
        core_axis_name="c", subcore_axis_name="s",
        num_cores=NC, num_subcores=NS,
    )
    return pl.kernel(
        _sc_edge_scatter_body,
        out_type=jax.ShapeDtypeStruct((NC, N_NODES, HIDDEN), F32),
        mesh=mesh,
        scratch_types=[
            pltpu.VMEM((2, IBLK, CH), jnp.int32),
            pltpu.VMEM((2, IBLK, CH), jnp.int32),
            pltpu.VMEM((CH, HIDDEN), F32),
            pltpu.VMEM((CH, HIDDEN), F32),
            pltpu.VMEM_SHARED((ACC_ROWS, HIDDEN), F32),
            pltpu.SemaphoreType.DMA,
            pltpu.SemaphoreType.DMA,
            pltpu.SemaphoreType.DMA,
            pltpu.SemaphoreType.DMA,
        ],
    )


def _sc_edge_scatter(msg, src_p, dst_p, zeros):
    return _sc_edge_scatter_kernel()(msg, src_p, dst_p, zeros)


def _lin_body(x_ref, w_ref, b_ref, o_ref):
    o_ref[...] = (
        jnp.dot(x_ref[...], w_ref[...], preferred_element_type=F32) + b_ref[...]
    )


def _gru(a, h, wih_ref, bih_ref, whh_ref, bhh_ref):
    gi = jnp.dot(a, wih_ref[...], preferred_element_type=F32) + bih_ref[...]
    gh = jnp.dot(h, whh_ref[...], preferred_element_type=F32) + bhh_ref[...]
    r = jax.nn.sigmoid(gi[:, :HIDDEN] + gh[:, :HIDDEN])
    z = jax.nn.sigmoid(gi[:, HIDDEN:2 * HIDDEN] + gh[:, HIDDEN:2 * HIDDEN])
    n = jnp.tanh(gi[:, 2 * HIDDEN:] + r * gh[:, 2 * HIDDEN:])
    return (1.0 - z) * n + z * h


def _gru_lin_body(a0_ref, a1_ref, h_ref, wih_ref, bih_ref, whh_ref, bhh_ref,
                  wlin_ref, blin_ref, h_out, msg_out):
    a = a0_ref[0] + a1_ref[0]
    hn = _gru(a, h_ref[...], wih_ref, bih_ref, whh_ref, bhh_ref)
    h_out[...] = hn
    msg_out[...] = (
        jnp.dot(hn, wlin_ref[...], preferred_element_type=F32) + blin_ref[...]
    )


def _gru_ln_body(a0_ref, a1_ref, h_ref, res_ref, wih_ref, bih_ref, whh_ref,
                 bhh_ref, gamma_ref, beta_ref, o_ref):
    a = a0_ref[0] + a1_ref[0]
    hn = _gru(a, h_ref[...], wih_ref, bih_ref, whh_ref, bhh_ref)
    x = hn + res_ref[...]
    mean = jnp.mean(x, axis=-1, keepdims=True)
    xc = x - mean
    var = jnp.mean(xc * xc, axis=-1, keepdims=True)
    y = xc * lax.rsqrt(var + 1e-5)
    y = y * gamma_ref[...] + beta_ref[...]
    o_ref[...] = jnp.where(y >= 0, y, 0.01 * y)


def _row_spec(shape=(ROW_BLK, HIDDEN)):
    return pl.BlockSpec(shape, lambda i: (0,) * (len(shape) - 2) + (i, 0))


def _full_spec(shape):
    return pl.BlockSpec(shape, lambda i: (0,) * len(shape))


_GRID = N_NODES // ROW_BLK


def _tc_linear(x, w, b):
    return pl.pallas_call(
        _lin_body,
        grid=(_GRID,),
        in_specs=[_row_spec(), _full_spec((HIDDEN, HIDDEN)), _full_spec((1, HIDDEN))],
        out_specs=_row_spec(),
        out_shape=jax.ShapeDtypeStruct((N_NODES, HIDDEN), F32),
    )(x, w, b)


def _tc_gru_lin(parts, h, wih, bih, whh, bhh, wlin, blin):
    part_spec = pl.BlockSpec((1, ROW_BLK, HIDDEN), lambda i: (0, i, 0))
    part_spec1 = pl.BlockSpec((1, ROW_BLK, HIDDEN), lambda i: (1, i, 0))
    return pl.pallas_call(
        _gru_lin_body,
        grid=(_GRID,),
        in_specs=[part_spec, part_spec1, _row_spec(),
                  _full_spec((HIDDEN, 3 * HIDDEN)), _full_spec((1, 3 * HIDDEN)),
                  _full_spec((HIDDEN, 3 * HIDDEN)), _full_spec((1, 3 * HIDDEN)),
                  _full_spec((HIDDEN, HIDDEN)), _full_spec((1, HIDDEN))],
        out_specs=[_row_spec(), _row_spec()],
        out_shape=[jax.ShapeDtypeStruct((N_NODES, HIDDEN), F32),
                   jax.ShapeDtypeStruct((N_NODES, HIDDEN), F32)],
    )(parts, parts, h, wih, bih, whh, bhh, wlin, blin)


def _tc_gru_ln(parts, h, res, wih, bih, whh, bhh, gamma, beta):
    part_spec = pl.BlockSpec((1, ROW_BLK, HIDDEN), lambda i: (0, i, 0))
    part_spec1 = pl.BlockSpec((1, ROW_BLK, HIDDEN), lambda i: (1, i, 0))
    return pl.pallas_call(
        _gru_ln_body,
        grid=(_GRID,),
        in_specs=[part_spec, part_spec1, _row_spec(), _row_spec(),
                  _full_spec((HIDDEN, 3 * HIDDEN)), _full_spec((1, 3 * HIDDEN)),
                  _full_spec((HIDDEN, 3 * HIDDEN)), _full_spec((1, 3 * HIDDEN)),
                  _full_spec((1, HIDDEN)), _full_spec((1, HIDDEN))],
        out_specs=_row_spec(),
        out_shape=jax.ShapeDtypeStruct((N_NODES, HIDDEN), F32),
    )(parts, parts, h, res, wih, bih, whh, bhh, gamma, beta)


def kernel(h_g2, edge_index, W_lin, b_lin, W_ih, b_ih, W_hh, b_hh, gamma, beta):
    src = edge_index[0].astype(jnp.int32)
    dst = edge_index[1].astype(jnp.int32)
    e = src.shape[0]
    src_p = jnp.concatenate(
        [src, jnp.zeros((E_PAD - e,), jnp.int32)]).reshape(TOT_CHUNKS, CH)
    dst_p = jnp.concatenate(
        [dst, jnp.full((E_PAD - e,), DUMMY, jnp.int32)]).reshape(TOT_CHUNKS, CH)
    zeros = jnp.zeros((ZROWS, HIDDEN), F32)

    w_lin_t = W_lin.T
    w_ih_t = W_ih.T
    w_hh_t = W_hh.T
    b_lin2 = b_lin.reshape(1, HIDDEN)
    b_ih2 = b_ih.reshape(1, 3 * HIDDEN)
    b_hh2 = b_hh.reshape(1, 3 * HIDDEN)
    gamma2 = gamma.reshape(1, HIDDEN)
    beta2 = beta.reshape(1, HIDDEN)

    msg = _tc_linear(h_g2, w_lin_t, b_lin2)
    parts = _sc_edge_scatter(msg, src_p, dst_p, zeros)
    feat, msg2 = _tc_gru_lin(parts, h_g2, w_ih_t, b_ih2, w_hh_t, b_hh2,
                             w_lin_t, b_lin2)
    parts2 = _sc_edge_scatter(msg2, src_p, dst_p, zeros)
    out = _tc_gru_ln(parts2, feat, h_g2, w_ih_t, b_ih2, w_hh_t, b_hh2,
                     gamma2, beta2)
    return out
